# baseline (device time: 268454 ns/iter reference)
import jax
import jax.numpy as jnp
from jax import lax
from jax.experimental import pallas as pl
from jax.experimental.pallas import tpu as pltpu

N_DEV = 8
SQ = 2048
DH = 128
HQ_LOC = 8
D_MODEL = 1024
QT = 512
KT = 512
N_QT = SQ // QT
SUB = QT // N_DEV
SCALE = 0.08838834764831843
BLK = 64
NEG = -1e9
_MESH = pl.DeviceIdType.MESH


def _fused_body(x_ref, wq_ref, kr_ref, vr_ref, wo_ref, out_ref,
                ctx_scr, kbuf, vbuf, tsend, ag_sbuf, rs_rbuf, ag_rbuf,
                copy_sem, rs_ssem, rs_rsem, ag_ssem, ag_rsem):
    my = lax.axis_index("i")
    col0 = my * (HQ_LOC * DH)

    barrier = pltpu.get_barrier_semaphore()
    for p in range(1, N_DEV):
        peer = lax.rem(my + p, N_DEV)
        pl.semaphore_signal(barrier, inc=1, device_id=(peer,),
                            device_id_type=_MESH)
    pl.semaphore_wait(barrier, N_DEV - 1)

    def kv_fetch(t, h, slot):
        n_kv = (t + 1) * KT
        cols = pl.ds(col0 + h * DH, DH)
        kc = pltpu.make_async_copy(
            kr_ref.at[pl.ds(0, n_kv), cols],
            kbuf.at[slot, pl.ds(0, n_kv), :], copy_sem.at[slot, 0])
        vc = pltpu.make_async_copy(
            vr_ref.at[pl.ds(0, n_kv), cols],
            vbuf.at[slot, pl.ds(0, n_kv), :], copy_sem.at[slot, 1])
        kc.start()
        vc.start()
        return kc, vc

    def compute_tile(t):
        n_kv = (t + 1) * KT
        xt = x_ref[t * QT:(t + 1) * QT, :]
        pending = kv_fetch(t, 0, 0)
        for h in range(HQ_LOC):
            slot = h % 2
            pending[0].wait()
            pending[1].wait()
            if h + 1 < HQ_LOC:
                pending = kv_fetch(t, h + 1, (h + 1) % 2)
            c0, c1 = h * DH, (h + 1) * DH
            q = jnp.dot(xt, wq_ref[:, c0:c1],
                        preferred_element_type=jnp.float32)
            q = (q * SCALE).astype(jnp.bfloat16)
            kh = kbuf[slot, 0:n_kv, :].astype(jnp.bfloat16)
            s = lax.dot_general(
                q, kh, (((1,), (1,)), ((), ())),
                preferred_element_type=jnp.float32,
            )
            row = t * QT + lax.broadcasted_iota(jnp.int32, (QT, n_kv), 0)
            col = lax.broadcasted_iota(jnp.int32, (QT, n_kv), 1)
            s = jnp.where((col // BLK) <= (row // BLK), s, NEG)
            m = jnp.max(s, axis=1, keepdims=True)
            w = jnp.exp(s - m)
            l_inv = 1.0 / jnp.sum(w, axis=1, keepdims=True)
            vh = vbuf[slot, 0:n_kv, :].astype(jnp.bfloat16)
            ctx = jnp.dot(w.astype(jnp.bfloat16), vh,
                          preferred_element_type=jnp.float32)
            ctx_scr[:, c0:c1] = (ctx * l_inv).astype(jnp.bfloat16)
        partial = jnp.dot(ctx_scr[...], wo_ref[...],
                          preferred_element_type=jnp.float32)
        tsend[t % 2] = partial.astype(jnp.bfloat16)

    def rs_descs(t):
        descs = []
        for j in range(N_DEV):
            q = lax.rem(my + j, N_DEV)
            descs.append(pltpu.make_async_remote_copy(
                src_ref=tsend.at[t % 2].at[pl.ds(q * SUB, SUB), :],
                dst_ref=rs_rbuf.at[t].at[pl.ds(my * SUB, SUB), :],
                send_sem=rs_ssem.at[t % 2, j],
                recv_sem=rs_rsem.at[t, my],
                device_id=(q,), device_id_type=_MESH,
            ))
        return descs

    def ag_descs(t):
        descs = []
        for j in range(N_DEV):
            q = lax.rem(my + j, N_DEV)
            descs.append(pltpu.make_async_remote_copy(
                src_ref=ag_sbuf.at[t % 2],
                dst_ref=ag_rbuf.at[t].at[pl.ds(my * SUB, SUB), :],
                send_sem=ag_ssem.at[t % 2, j],
                recv_sem=ag_rsem.at[t, my],
                device_id=(q,), device_id_type=_MESH,
            ))
        return descs

    def wait_recvs(t, rbuf, rsem):
        for s in range(N_DEV):
            desc = pltpu.make_async_remote_copy(
                src_ref=tsend.at[0].at[pl.ds(0, SUB), :],
                dst_ref=rbuf.at[t].at[pl.ds(s * SUB, SUB), :],
                send_sem=rs_ssem.at[0, 0],
                recv_sem=rsem.at[t, s],
                device_id=(my,), device_id_type=_MESH,
            )
            desc.wait_recv()

    rs_inflight = {}
    ag_inflight = {}

    def finish_rs(t):
        wait_recvs(t, rs_rbuf, rs_rsem)
        red = jnp.zeros((SUB, D_MODEL), jnp.float32)
        for s in range(N_DEV):
            red = red + rs_rbuf[t, s * SUB:(s + 1) * SUB, :].astype(
                jnp.float32)
        ag_sbuf[t % 2] = red.astype(jnp.bfloat16)
        ag_inflight[t] = ag_descs(t)
        for d in ag_inflight[t]:
            d.start()
        for d in rs_inflight[t]:
            d.wait_send()

    def finish_ag(t):
        wait_recvs(t, ag_rbuf, ag_rsem)
        out_ref[0, t * QT:(t + 1) * QT, :] = ag_rbuf[t].astype(jnp.float32)
        for d in ag_inflight[t]:
            d.wait_send()

    for t in range(N_QT):
        compute_tile(t)
        rs_inflight[t] = rs_descs(t)
        for d in rs_inflight[t]:
            d.start()
        if t >= 1:
            finish_rs(t - 1)
        if t >= 2:
            finish_ag(t - 2)
    finish_rs(N_QT - 1)
    finish_ag(N_QT - 2)
    finish_ag(N_QT - 1)


def kernel(x, Wq, K_ext, V_ext, Wo):
    x2 = x[0].astype(jnp.bfloat16)
    wq = Wq.astype(jnp.bfloat16)
    kr = K_ext[0].reshape(SQ, 64 * DH)
    vr = V_ext[0].reshape(SQ, 64 * DH)
    wo = Wo.astype(jnp.bfloat16)

    out = pl.pallas_call(
        _fused_body,
        in_specs=[
            pl.BlockSpec(memory_space=pltpu.VMEM),
            pl.BlockSpec(memory_space=pltpu.VMEM),
            pl.BlockSpec(memory_space=pltpu.MemorySpace.HBM),
            pl.BlockSpec(memory_space=pltpu.MemorySpace.HBM),
            pl.BlockSpec(memory_space=pltpu.VMEM),
        ],
        out_specs=pl.BlockSpec(memory_space=pltpu.VMEM),
        out_shape=jax.ShapeDtypeStruct((1, SQ, D_MODEL), jnp.float32),
        scratch_shapes=[
            pltpu.VMEM((QT, D_MODEL), jnp.bfloat16),
            pltpu.VMEM((2, SQ, DH), jnp.float32),
            pltpu.VMEM((2, SQ, DH), jnp.float32),
            pltpu.VMEM((2, QT, D_MODEL), jnp.bfloat16),
            pltpu.VMEM((2, SUB, D_MODEL), jnp.bfloat16),
            pltpu.VMEM((N_QT, QT, D_MODEL), jnp.bfloat16),
            pltpu.VMEM((N_QT, QT, D_MODEL), jnp.bfloat16),
            pltpu.SemaphoreType.DMA((2, 2)),
            pltpu.SemaphoreType.DMA((2, N_DEV)),
            pltpu.SemaphoreType.DMA((N_QT, N_DEV)),
            pltpu.SemaphoreType.DMA((2, N_DEV)),
            pltpu.SemaphoreType.DMA((N_QT, N_DEV)),
        ],
        compiler_params=pltpu.CompilerParams(collective_id=0),
    )(x2, wq, kr, vr, wo)

    return out


# device time: 118067 ns/iter; 2.2737x vs baseline; 2.2737x over previous
import jax
import jax.numpy as jnp
from jax import lax
from jax.experimental import pallas as pl
from jax.experimental.pallas import tpu as pltpu

N_DEV = 8
SQ = 2048
DH = 128
HQ_LOC = 8
D_MODEL = 1024
QT = 512
KT = 512
N_QT = SQ // QT
SUB = QT // N_DEV
SCALE = 0.08838834764831843
BLK = 64
NEG = -1e9
_MESH = pl.DeviceIdType.MESH


def _fused_body(x_ref, wq_ref, kr_ref, vr_ref, wo_ref, out_ref,
                ctx_scr, kbuf, vbuf, tsend, ag_sbuf, rs_rbuf, ag_rbuf,
                copy_sem, rs_ssem, rs_rsem, ag_ssem, ag_rsem):
    my = lax.axis_index("i")
    h0 = my * HQ_LOC

    barrier = pltpu.get_barrier_semaphore()
    for p in range(1, N_DEV):
        peer = lax.rem(my + p, N_DEV)
        pl.semaphore_signal(barrier, inc=1, device_id=(peer,),
                            device_id_type=_MESH)
    pl.semaphore_wait(barrier, N_DEV - 1)

    def kv_fetch(t, h, slot):
        n_kv = (t + 1) * KT
        gh = h0 + h
        kc = pltpu.make_async_copy(
            kr_ref.at[pl.ds(0, n_kv), gh, :],
            kbuf.at[slot, pl.ds(0, n_kv), :], copy_sem.at[slot, 0])
        vc = pltpu.make_async_copy(
            vr_ref.at[pl.ds(0, n_kv), gh, :],
            vbuf.at[slot, pl.ds(0, n_kv), :], copy_sem.at[slot, 1])
        kc.start()
        vc.start()
        return kc, vc

    def compute_tile(t):
        n_kv = (t + 1) * KT
        xt = x_ref[t * QT:(t + 1) * QT, :]
        pending = kv_fetch(t, 0, 0)
        for h in range(HQ_LOC):
            slot = h % 2
            pending[0].wait()
            pending[1].wait()
            if h + 1 < HQ_LOC:
                pending = kv_fetch(t, h + 1, (h + 1) % 2)
            c0, c1 = h * DH, (h + 1) * DH
            q = jnp.dot(xt, wq_ref[:, c0:c1],
                        preferred_element_type=jnp.float32)
            q = (q * SCALE).astype(jnp.bfloat16)
            kh = kbuf[slot, 0:n_kv, :].astype(jnp.bfloat16)
            s = lax.dot_general(
                q, kh, (((1,), (1,)), ((), ())),
                preferred_element_type=jnp.float32,
            )
            row = t * QT + lax.broadcasted_iota(jnp.int32, (QT, n_kv), 0)
            col = lax.broadcasted_iota(jnp.int32, (QT, n_kv), 1)
            s = jnp.where((col // BLK) <= (row // BLK), s, NEG)
            m = jnp.max(s, axis=1, keepdims=True)
            w = jnp.exp(s - m)
            l_inv = 1.0 / jnp.sum(w, axis=1, keepdims=True)
            vh = vbuf[slot, 0:n_kv, :].astype(jnp.bfloat16)
            ctx = jnp.dot(w.astype(jnp.bfloat16), vh,
                          preferred_element_type=jnp.float32)
            ctx_scr[:, c0:c1] = (ctx * l_inv).astype(jnp.bfloat16)
        partial = jnp.dot(ctx_scr[...], wo_ref[...],
                          preferred_element_type=jnp.float32)
        tsend[t % 2] = partial.astype(jnp.bfloat16)

    def rs_descs(t):
        descs = []
        for j in range(N_DEV):
            q = lax.rem(my + j, N_DEV)
            descs.append(pltpu.make_async_remote_copy(
                src_ref=tsend.at[t % 2].at[pl.ds(q * SUB, SUB), :],
                dst_ref=rs_rbuf.at[t].at[pl.ds(my * SUB, SUB), :],
                send_sem=rs_ssem.at[t % 2, j],
                recv_sem=rs_rsem.at[t, my],
                device_id=(q,), device_id_type=_MESH,
            ))
        return descs

    def ag_descs(t):
        descs = []
        for j in range(N_DEV):
            q = lax.rem(my + j, N_DEV)
            descs.append(pltpu.make_async_remote_copy(
                src_ref=ag_sbuf.at[t % 2],
                dst_ref=ag_rbuf.at[t].at[pl.ds(my * SUB, SUB), :],
                send_sem=ag_ssem.at[t % 2, j],
                recv_sem=ag_rsem.at[t, my],
                device_id=(q,), device_id_type=_MESH,
            ))
        return descs

    def wait_recvs(t, rbuf, rsem):
        for s in range(N_DEV):
            desc = pltpu.make_async_remote_copy(
                src_ref=tsend.at[0].at[pl.ds(0, SUB), :],
                dst_ref=rbuf.at[t].at[pl.ds(s * SUB, SUB), :],
                send_sem=rs_ssem.at[0, 0],
                recv_sem=rsem.at[t, s],
                device_id=(my,), device_id_type=_MESH,
            )
            desc.wait_recv()

    rs_inflight = {}
    ag_inflight = {}

    def finish_rs(t):
        wait_recvs(t, rs_rbuf, rs_rsem)
        red = jnp.zeros((SUB, D_MODEL), jnp.float32)
        for s in range(N_DEV):
            red = red + rs_rbuf[t, s * SUB:(s + 1) * SUB, :].astype(
                jnp.float32)
        ag_sbuf[t % 2] = red.astype(jnp.bfloat16)
        ag_inflight[t] = ag_descs(t)
        for d in ag_inflight[t]:
            d.start()
        for d in rs_inflight[t]:
            d.wait_send()

    def finish_ag(t):
        wait_recvs(t, ag_rbuf, ag_rsem)
        out_ref[0, t * QT:(t + 1) * QT, :] = ag_rbuf[t].astype(jnp.float32)
        for d in ag_inflight[t]:
            d.wait_send()

    for t in range(N_QT):
        compute_tile(t)
        rs_inflight[t] = rs_descs(t)
        for d in rs_inflight[t]:
            d.start()
        if t >= 1:
            finish_rs(t - 1)
        if t >= 2:
            finish_ag(t - 2)
    finish_rs(N_QT - 1)
    finish_ag(N_QT - 2)
    finish_ag(N_QT - 1)


def kernel(x, Wq, K_ext, V_ext, Wo):
    x2 = x[0].astype(jnp.bfloat16)
    wq = Wq.astype(jnp.bfloat16)
    kr = K_ext[0]
    vr = V_ext[0]
    wo = Wo.astype(jnp.bfloat16)

    out = pl.pallas_call(
        _fused_body,
        in_specs=[
            pl.BlockSpec(memory_space=pltpu.VMEM),
            pl.BlockSpec(memory_space=pltpu.VMEM),
            pl.BlockSpec(memory_space=pltpu.MemorySpace.HBM),
            pl.BlockSpec(memory_space=pltpu.MemorySpace.HBM),
            pl.BlockSpec(memory_space=pltpu.VMEM),
        ],
        out_specs=pl.BlockSpec(memory_space=pltpu.VMEM),
        out_shape=jax.ShapeDtypeStruct((1, SQ, D_MODEL), jnp.float32),
        scratch_shapes=[
            pltpu.VMEM((QT, D_MODEL), jnp.bfloat16),
            pltpu.VMEM((2, SQ, DH), jnp.float32),
            pltpu.VMEM((2, SQ, DH), jnp.float32),
            pltpu.VMEM((2, QT, D_MODEL), jnp.bfloat16),
            pltpu.VMEM((2, SUB, D_MODEL), jnp.bfloat16),
            pltpu.VMEM((N_QT, QT, D_MODEL), jnp.bfloat16),
            pltpu.VMEM((N_QT, QT, D_MODEL), jnp.bfloat16),
            pltpu.SemaphoreType.DMA((2, 2)),
            pltpu.SemaphoreType.DMA((2, N_DEV)),
            pltpu.SemaphoreType.DMA((N_QT, N_DEV)),
            pltpu.SemaphoreType.DMA((2, N_DEV)),
            pltpu.SemaphoreType.DMA((N_QT, N_DEV)),
        ],
        compiler_params=pltpu.CompilerParams(collective_id=0),
    )(x2, wq, kr, vr, wo)

    return out


# device time: 112306 ns/iter; 2.3904x vs baseline; 1.0513x over previous
import jax
import jax.numpy as jnp
from jax import lax
from jax.experimental import pallas as pl
from jax.experimental.pallas import tpu as pltpu

N_DEV = 8
SQ = 2048
DH = 128
HQ_LOC = 8
D_MODEL = 1024
QT = 512
KT = 512
N_QT = SQ // QT
SUB = QT // N_DEV
SCALE = 0.08838834764831843
BLK = 64
NEG = -1e9
_MESH = pl.DeviceIdType.MESH


def _fused_body(x_ref, wq_ref, k_ref, v_ref, wo_ref, out_ref,
                ctx_scr, tsend, ag_sbuf, rs_rbuf, ag_rbuf,
                rs_ssem, rs_rsem, ag_ssem, ag_rsem):
    my = lax.axis_index("i")

    barrier = pltpu.get_barrier_semaphore()
    for p in range(1, N_DEV):
        peer = lax.rem(my + p, N_DEV)
        pl.semaphore_signal(barrier, inc=1, device_id=(peer,),
                            device_id_type=_MESH)
    pl.semaphore_wait(barrier, N_DEV - 1)

    def compute_tile(t):
        n_kv = (t + 1) * KT
        xt = x_ref[t * QT:(t + 1) * QT, :]
        for h in range(HQ_LOC):
            c0, c1 = h * DH, (h + 1) * DH
            q = jnp.dot(xt, wq_ref[:, c0:c1],
                        preferred_element_type=jnp.float32)
            q = (q * SCALE).astype(jnp.bfloat16)
            s = lax.dot_general(
                q, k_ref[0:n_kv, c0:c1], (((1,), (1,)), ((), ())),
                preferred_element_type=jnp.float32,
            )
            row = t * QT + lax.broadcasted_iota(jnp.int32, (QT, n_kv), 0)
            col = lax.broadcasted_iota(jnp.int32, (QT, n_kv), 1)
            s = jnp.where((col // BLK) <= (row // BLK), s, NEG)
            m = jnp.max(s, axis=1, keepdims=True)
            w = jnp.exp(s - m)
            w = w / jnp.sum(w, axis=1, keepdims=True)
            ctx_scr[:, c0:c1] = jnp.dot(
                w.astype(jnp.bfloat16), v_ref[0:n_kv, c0:c1],
                preferred_element_type=jnp.float32,
            ).astype(jnp.bfloat16)
        partial = jnp.dot(ctx_scr[...], wo_ref[...],
                          preferred_element_type=jnp.float32)
        tsend[t % 2] = partial.astype(jnp.bfloat16)

    def rs_descs(t):
        descs = []
        for j in range(N_DEV):
            q = lax.rem(my + j, N_DEV)
            descs.append(pltpu.make_async_remote_copy(
                src_ref=tsend.at[t % 2].at[pl.ds(q * SUB, SUB), :],
                dst_ref=rs_rbuf.at[t].at[pl.ds(my * SUB, SUB), :],
                send_sem=rs_ssem.at[t % 2, j],
                recv_sem=rs_rsem.at[t, my],
                device_id=(q,), device_id_type=_MESH,
            ))
        return descs

    def ag_descs(t):
        descs = []
        for j in range(N_DEV):
            q = lax.rem(my + j, N_DEV)
            descs.append(pltpu.make_async_remote_copy(
                src_ref=ag_sbuf.at[t % 2],
                dst_ref=ag_rbuf.at[t].at[pl.ds(my * SUB, SUB), :],
                send_sem=ag_ssem.at[t % 2, j],
                recv_sem=ag_rsem.at[t, my],
                device_id=(q,), device_id_type=_MESH,
            ))
        return descs

    def wait_recvs(t, rbuf, rsem):
        for s in range(N_DEV):
            desc = pltpu.make_async_remote_copy(
                src_ref=tsend.at[0].at[pl.ds(0, SUB), :],
                dst_ref=rbuf.at[t].at[pl.ds(s * SUB, SUB), :],
                send_sem=rs_ssem.at[0, 0],
                recv_sem=rsem.at[t, s],
                device_id=(my,), device_id_type=_MESH,
            )
            desc.wait_recv()

    rs_inflight = {}
    ag_inflight = {}

    def finish_rs(t):
        wait_recvs(t, rs_rbuf, rs_rsem)
        red = jnp.zeros((SUB, D_MODEL), jnp.float32)
        for s in range(N_DEV):
            red = red + rs_rbuf[t, s * SUB:(s + 1) * SUB, :].astype(
                jnp.float32)
        ag_sbuf[t % 2] = red.astype(jnp.bfloat16)
        ag_inflight[t] = ag_descs(t)
        for d in ag_inflight[t]:
            d.start()
        for d in rs_inflight[t]:
            d.wait_send()

    def finish_ag(t):
        wait_recvs(t, ag_rbuf, ag_rsem)
        out_ref[t * QT:(t + 1) * QT, :] = ag_rbuf[t].astype(jnp.float32)
        for d in ag_inflight[t]:
            d.wait_send()

    for t in range(N_QT):
        compute_tile(t)
        rs_inflight[t] = rs_descs(t)
        for d in rs_inflight[t]:
            d.start()
        if t >= 1:
            finish_rs(t - 1)
        if t >= 2:
            finish_ag(t - 2)
    finish_rs(N_QT - 1)
    finish_ag(N_QT - 2)
    finish_ag(N_QT - 1)


def kernel(x, Wq, K_ext, V_ext, Wo):
    my = lax.axis_index("i")
    x2 = x[0].astype(jnp.bfloat16)
    wq = Wq.astype(jnp.bfloat16)
    k = lax.dynamic_slice_in_dim(
        K_ext[0], my * HQ_LOC, HQ_LOC, axis=1
    ).astype(jnp.bfloat16).reshape(SQ, HQ_LOC * DH)
    v = lax.dynamic_slice_in_dim(
        V_ext[0], my * HQ_LOC, HQ_LOC, axis=1
    ).astype(jnp.bfloat16).reshape(SQ, HQ_LOC * DH)
    wo = Wo.astype(jnp.bfloat16)

    out = pl.pallas_call(
        _fused_body,
        in_specs=[pl.BlockSpec(memory_space=pltpu.VMEM)] * 5,
        out_specs=pl.BlockSpec(memory_space=pltpu.VMEM),
        out_shape=jax.ShapeDtypeStruct((SQ, D_MODEL), jnp.float32),
        scratch_shapes=[
            pltpu.VMEM((QT, D_MODEL), jnp.bfloat16),
            pltpu.VMEM((2, QT, D_MODEL), jnp.bfloat16),
            pltpu.VMEM((2, SUB, D_MODEL), jnp.bfloat16),
            pltpu.VMEM((N_QT, QT, D_MODEL), jnp.bfloat16),
            pltpu.VMEM((N_QT, QT, D_MODEL), jnp.bfloat16),
            pltpu.SemaphoreType.DMA((2, N_DEV)),
            pltpu.SemaphoreType.DMA((N_QT, N_DEV)),
            pltpu.SemaphoreType.DMA((2, N_DEV)),
            pltpu.SemaphoreType.DMA((N_QT, N_DEV)),
        ],
        compiler_params=pltpu.CompilerParams(collective_id=0),
    )(x2, wq, k, v, wo)

    return out[None]
